# Initial kernel scaffold; baseline (speedup 1.0000x reference)
import jax
import jax.numpy as jnp
from jax.experimental import pallas as pl


def kernel(inputs, codebook):
    raise NotImplementedError("write your pallas kernel here")


# speedup vs baseline: 1.1021x; 1.1021x over previous
import functools

import jax
import jax.numpy as jnp
from jax import lax
from jax.experimental import pallas as pl
from jax.experimental.pallas import tpu as pltpu
from jax.experimental.pallas import tpu_sc as plsc


def _argmin_body(x_ref, cb_ref, idx_ref):
    x = x_ref[...]
    cb = cb_ref[...]
    x_sq = jnp.sum(x * x, axis=-1, keepdims=True)
    c_sq = jnp.sum(cb * cb, axis=-1)
    dots = lax.dot_general(x, cb, (((1,), (1,)), ((), ())),
                           preferred_element_type=jnp.float32)
    d2 = x_sq + c_sq[None, :] - 2.0 * dots
    l2 = jnp.sqrt(jnp.maximum(d2, 0.0))
    idx = jnp.argmin(l2, axis=-1).astype(jnp.int32)
    idx_ref[...] = idx[None, None, :]


def _nearest_indices(x2d, codebook, tile_m):
    n, d = x2d.shape
    k = codebook.shape[0]
    grid = n // tile_m
    return pl.pallas_call(
        _argmin_body,
        grid=(grid,),
        in_specs=[
            pl.BlockSpec((tile_m, d), lambda i: (i, 0)),
            pl.BlockSpec((k, d), lambda i: (0, 0)),
        ],
        out_specs=pl.BlockSpec((1, 1, tile_m), lambda i: (i, 0, 0)),
        out_shape=jax.ShapeDtypeStruct((grid, 1, tile_m), jnp.int32),
    )(x2d, codebook)


def _make_sc_gather(n, d):
    info = plsc.get_sparse_core_info()
    nw = info.num_cores * info.num_subcores
    b_per_w = n // nw
    mesh = plsc.VectorSubcoreMesh(core_axis_name="c", subcore_axis_name="s")

    @functools.partial(
        pl.kernel, mesh=mesh,
        out_type=jax.ShapeDtypeStruct((n, d), jnp.float32),
        compiler_params=pltpu.CompilerParams(use_tc_tiling_on_sc=False),
        scratch_types=[
            pltpu.VMEM((b_per_w,), jnp.int32),
            pltpu.VMEM((b_per_w, d), jnp.float32),
            pltpu.SemaphoreType.DMA,
        ],
    )
    def gather(table_hbm, idx_hbm, out_hbm, idx_v, rows_v, sem):
        wid = lax.axis_index("s") * info.num_cores + lax.axis_index("c")
        base = wid * b_per_w
        pltpu.sync_copy(idx_hbm.at[pl.ds(base, b_per_w)], idx_v)
        pltpu.async_copy(table_hbm.at[idx_v], rows_v, sem).wait()
        pltpu.sync_copy(rows_v, out_hbm.at[pl.ds(base, b_per_w)])

    return gather


def kernel(inputs, codebook):
    b, t, d = inputs.shape
    n = b * t
    x2d = inputs.reshape(n, d)
    idx = _nearest_indices(x2d, codebook, tile_m=2048).reshape(n)
    quantized = _make_sc_gather(n, d)(codebook, idx)
    return quantized.reshape(b, t, d)
